# baseline (speedup 1.0000x reference)
import functools

import jax
import jax.numpy as jnp
from jax import lax
from jax.experimental import pallas as pl
from jax.experimental.pallas import tpu as pltpu
from jax.experimental.pallas import tpu_sc as plsc

D_MODEL = 768
N_TOK = 4 * 8192
NC, NS, L = 2, 16, 16
NW = NC * NS
TOK_W = N_TOK // NW
C = 16
NCH = TOK_W // C
NR = 2
NO = 4

_mesh = plsc.VectorSubcoreMesh(core_axis_name="c", subcore_axis_name="s")


@functools.partial(
    pl.kernel,
    out_type=jax.ShapeDtypeStruct((N_TOK, D_MODEL), jnp.float32),
    mesh=_mesh,
    scratch_types=(
        [pltpu.VMEM((NCH, C), jnp.int32)]
        + [pltpu.VMEM((C, D_MODEL), jnp.float32) for _ in range(NR + NO)]
        + [pltpu.SemaphoreType.DMA for _ in range(NR + 2 * NO)]
    ),
)
def _pe_kernel(x_hbm, pos_hbm, tbl_hbm, out_hbm,
               idx_v, rb0, rb1, ob0, ob1, ob2, ob3,
               sr0, sr1, sx0, sx1, sx2, sx3, so0, so1, so2, so3):
    cid = lax.axis_index("c")
    sid = lax.axis_index("s")
    wid = sid * NC + cid
    base = wid * TOK_W

    rbs, srs = (rb0, rb1), (sr0, sr1)
    obs, sxs, sos = (ob0, ob1, ob2, ob3), (sx0, sx1, sx2, sx3), (so0, so1, so2, so3)

    pltpu.sync_copy(pos_hbm.at[wid], idx_v)

    def fire_x(c, bo):
        pltpu.async_copy(x_hbm.at[pl.ds(base + c * C, C)], obs[bo], sxs[bo])

    def fire_gather(c, br):
        pltpu.async_copy(tbl_hbm.at[idx_v.at[c]], rbs[br], srs[br])

    fire_x(0, 0)
    fire_x(1, 1)
    fire_gather(0, 0)
    fire_gather(1, 1)

    def outer(g, carry):
        for b in range(NO):
            c = NO * g + b
            br = b % NR
            bo = b
            pltpu.make_async_copy(x_hbm.at[pl.ds(0, C)], obs[bo], sxs[bo]).wait()
            pltpu.make_async_copy(x_hbm.at[pl.ds(0, C)], rbs[br], srs[br]).wait()

            def add_row(t, acc):
                for j in range(D_MODEL // L):
                    sl = pl.ds(j * L, L)
                    plsc.addupdate(obs[bo].at[t, sl], rbs[br][t, sl])
                return acc

            lax.fori_loop(0, C, add_row, 0)

            pltpu.async_copy(obs[bo], out_hbm.at[pl.ds(base + c * C, C)], sos[bo])

            @pl.when(c + 2 < NCH)
            def _():
                fire_gather(c + 2, br)

            bo2 = (b + 2) % NO
            @pl.when(c >= 2)
            def _():
                pltpu.make_async_copy(
                    x_hbm.at[pl.ds(0, C)], obs[bo2], sos[bo2]).wait()

            @pl.when(c + 2 < NCH)
            def _():
                fire_x(c + 2, bo2)
        return carry

    lax.fori_loop(0, NCH // NO, outer, 0)

    for b in ((NCH - 2) % NO, (NCH - 1) % NO):
        pltpu.make_async_copy(x_hbm.at[pl.ds(0, C)], obs[b], sos[b]).wait()


def kernel(x, pos, pos_embedding):
    x2 = x.reshape(N_TOK, D_MODEL)
    idx = pos.astype(jnp.int32).reshape(NW, NCH, C)
    out = _pe_kernel(x2, idx, pos_embedding)
    return out.reshape(x.shape)
